# baseline (speedup 1.0000x reference)
import functools

import jax
import jax.numpy as jnp
from jax import lax
from jax.experimental import pallas as pl
from jax.experimental.pallas import tpu as pltpu
from jax.experimental.pallas import tpu_sc as plsc

_TB = 1024


def _sc_gather(table, idx_flat):
    _, H = table.shape
    N = idx_flat.shape[0]
    NC, NS = 2, 16
    NW = NC * NS
    rows_per_w = N // NW
    CH = 128
    n_ch = rows_per_w // CH

    mesh = plsc.VectorSubcoreMesh(core_axis_name="c", subcore_axis_name="s")

    @functools.partial(
        pl.kernel,
        mesh=mesh,
        out_type=jax.ShapeDtypeStruct((N, H), jnp.float32),
        scratch_types=[
            pltpu.VMEM((CH,), jnp.int32),
            pltpu.VMEM((CH, H), jnp.float32),
            pltpu.SemaphoreType.DMA,
        ],
    )
    def k(table_hbm, idx_hbm, out_hbm, idx_v, rows_v, sem):
        wid = lax.axis_index("s") * NC + lax.axis_index("c")
        base = wid * rows_per_w
        for c in range(n_ch):
            off = base + c * CH
            pltpu.sync_copy(idx_hbm.at[pl.ds(off, CH)], idx_v)
            pltpu.async_copy(table_hbm.at[idx_v], rows_v, sem).wait()
            pltpu.sync_copy(rows_v, out_hbm.at[pl.ds(off, CH)])

    return k(table, idx_flat)


def _char_body(TB, W, CV, c_ref, ce_ref, cw_ref, cb_ref, o_ref):
    H = cw_ref.shape[1]
    cid2 = c_ref[...].reshape(TB, W).astype(jnp.bfloat16)
    col = lax.broadcasted_iota(jnp.int32, (TB, CV), 1).astype(jnp.bfloat16)
    one = jnp.ones((TB, CV), jnp.bfloat16)
    zero = jnp.zeros((TB, CV), jnp.bfloat16)
    oh = jnp.concatenate(
        [jnp.where(col == cid2[:, w:w + 1], one, zero) for w in range(W)],
        axis=0)
    ce = jnp.dot(oh, ce_ref[...].astype(jnp.bfloat16),
                 preferred_element_type=jnp.float32).astype(jnp.float8_e4m3fn)
    CD = ce.shape[1]
    ce3 = ce.reshape(W, TB, CD)
    z = jnp.zeros((1, TB, CD), jnp.float8_e4m3fn)
    prev = jnp.concatenate([z, ce3[: W - 1]], axis=0)
    nxt = jnp.concatenate([ce3[1:], z], axis=0)
    x3 = jnp.concatenate([prev, ce3, nxt], axis=2)
    cw = cw_ref[...].astype(jnp.float8_e4m3fn)
    G = 2
    acc = None
    for g in range(0, W, G):
        yg = jnp.dot(x3[g:g + G].reshape(G * TB, x3.shape[2]), cw,
                     preferred_element_type=jnp.float32)
        m = jnp.max(yg.reshape(G, TB, H), axis=0)
        acc = m if acc is None else jnp.maximum(acc, m)
    cf = jnp.maximum(acc + cb_ref[...], 0.0)
    o_ref[...] = cf.astype(jnp.bfloat16)


def _char_feat(cids3, ce16, cw16, cb2):
    NB, TB, W = cids3.shape
    CV, CD = ce16.shape
    H = cw16.shape[1]
    body = functools.partial(_char_body, TB, W, CV)
    return pl.pallas_call(
        body,
        grid=(NB,),
        in_specs=[
            pl.BlockSpec((1, TB, W), lambda i: (i, 0, 0)),
            pl.BlockSpec((CV, CD), lambda i: (0, 0)),
            pl.BlockSpec((3 * CD, H), lambda i: (0, 0)),
            pl.BlockSpec((1, H), lambda i: (0, 0)),
        ],
        out_specs=pl.BlockSpec((TB, H), lambda i: (i, 0)),
        out_shape=jax.ShapeDtypeStruct((NB * TB, H), jnp.bfloat16),
        compiler_params=pltpu.CompilerParams(
            dimension_semantics=("arbitrary",)),
    )(cids3, ce16, cw16, cb2)


def _main_body(we_ref, pos_ref, ttf_ref, type_ref, cf_ref, aug_ref,
               augw_ref, augb_ref, clw_ref, clb_ref, g_ref, b_ref, o_ref):
    H = we_ref.shape[1]
    TB = we_ref.shape[0]
    L = pos_ref.shape[0]
    ttf = ttf_ref[...].reshape(TB, 1).astype(jnp.float32)
    t0 = type_ref[0:1, :]
    t1 = type_ref[1:2, :]
    pos = pos_ref[...]
    if TB > L:
        pos = jnp.concatenate([pos] * (TB // L), axis=0)
    emb = we_ref[...] + pos + t0 + ttf * (t1 - t0)
    h = (jnp.dot(emb.astype(jnp.bfloat16), clw_ref[0:H, :],
                 preferred_element_type=jnp.float32)
         + jnp.dot(cf_ref[...], clw_ref[H:2 * H, :],
                   preferred_element_type=jnp.float32)
         + clb_ref[...])
    h = h + jnp.dot(aug_ref[...], augw_ref[...],
                    preferred_element_type=jnp.float32) + augb_ref[...]
    mean = jnp.mean(h, axis=1, keepdims=True)
    d = h - mean
    var = jnp.mean(d * d, axis=1, keepdims=True)
    o_ref[...] = d * lax.rsqrt(var + 1e-12) * g_ref[...] + b_ref[...]


def _main(we, pos_emb, ttf, type_emb, cf16, aug_in, aug_w, augb2, clw16,
          clb2, g2, b2, L):
    N, H = we.shape
    TB = _TB
    AD = aug_w.shape[0]
    return pl.pallas_call(
        _main_body,
        grid=(N // TB,),
        in_specs=[
            pl.BlockSpec((TB, H), lambda i: (i, 0)),
            pl.BlockSpec((L, H), lambda i: (0, 0)),
            pl.BlockSpec((1, 1, TB), lambda i: (i, 0, 0)),
            pl.BlockSpec((2, H), lambda i: (0, 0)),
            pl.BlockSpec((TB, H), lambda i: (i, 0)),
            pl.BlockSpec((TB, AD), lambda i: (i, 0)),
            pl.BlockSpec((AD, H), lambda i: (0, 0)),
            pl.BlockSpec((1, H), lambda i: (0, 0)),
            pl.BlockSpec((2 * H, H), lambda i: (0, 0)),
            pl.BlockSpec((1, H), lambda i: (0, 0)),
            pl.BlockSpec((1, H), lambda i: (0, 0)),
            pl.BlockSpec((1, H), lambda i: (0, 0)),
        ],
        out_specs=pl.BlockSpec((TB, H), lambda i: (i, 0)),
        out_shape=jax.ShapeDtypeStruct((N, H), jnp.float32),
        compiler_params=pltpu.CompilerParams(
            dimension_semantics=("arbitrary",)),
    )(we, pos_emb, ttf, type_emb, cf16, aug_in, aug_w, augb2, clw16, clb2,
      g2, b2)


def kernel(char_input_ids, sent_token_aug, input_ids, token_type_ids,
           word_emb, pos_emb, type_emb, char_emb, conv_w, conv_b,
           char_lin_w, char_lin_b, aug_w, aug_b, gamma, beta):
    B, L = input_ids.shape
    W = char_input_ids.shape[-1]
    H = word_emb.shape[1]
    AD = sent_token_aug.shape[-1]
    N = B * L

    we = _sc_gather(word_emb, input_ids.reshape(N).astype(jnp.int32))

    cf16 = _char_feat(
        char_input_ids.reshape(N // _TB, _TB, W).astype(jnp.int32),
        char_emb,
        conv_w.reshape(3 * char_emb.shape[1], H),
        conv_b.reshape(1, H),
    )

    out = _main(
        we,
        pos_emb,
        token_type_ids.reshape(N // _TB, 1, _TB).astype(jnp.int32),
        type_emb,
        cf16,
        sent_token_aug.reshape(N, AD),
        aug_w,
        aug_b.reshape(1, H),
        char_lin_w.astype(jnp.bfloat16),
        char_lin_b.reshape(1, H),
        gamma.reshape(1, H),
        beta.reshape(1, H),
        L,
    )
    return out.reshape(B, L, H)
